# baseline (speedup 1.0000x reference)
import functools

import jax
import jax.numpy as jnp
from jax import lax
from jax.experimental import pallas as pl
from jax.experimental.pallas import tpu as pltpu
from jax.experimental.pallas import tpu_sc as plsc

D = 64
B = 16384
L = 200
LANES = 16

NC = 2
NS = 16
NW = NC * NS
ROWS_PER_W = B // NW

G0, G1 = 96, 104

IDS_AHEAD, NIDS = 5, 6
G_AHEAD, NG = 3, 4
UNROLL = 12
N_STEADY = ((ROWS_PER_W - IDS_AHEAD - G_AHEAD) // UNROLL) * UNROLL
TAIL_START = G_AHEAD + N_STEADY


def _issue_ids(tok_hbm, b, ids_v, sem):
    pltpu.async_copy(tok_hbm.at[pl.ds(b * L, L)], ids_v.at[pl.ds(0, L)], sem)


def _wait_ids(tok_hbm, b, ids_v, sem):
    pltpu.make_async_copy(
        tok_hbm.at[pl.ds(b * L, L)], ids_v.at[pl.ds(0, L)], sem).wait()


def _issue_gather(tab_hbm, ids_v, rows_v, sem_a, sem_b):
    pltpu.async_copy(
        tab_hbm.at[ids_v.at[pl.ds(0, G0)]], rows_v.at[pl.ds(0, G0)], sem_a)
    pltpu.async_copy(
        tab_hbm.at[ids_v.at[pl.ds(G0, G1)]], rows_v.at[pl.ds(G0, G1)], sem_b)


def _wait_gather_a(tab_hbm, ids_v, rows_v, sem_a):
    pltpu.make_async_copy(
        tab_hbm.at[ids_v.at[pl.ds(0, G0)]], rows_v.at[pl.ds(0, G0)],
        sem_a).wait()


def _wait_gather_b(tab_hbm, ids_v, rows_v, sem_b):
    pltpu.make_async_copy(
        tab_hbm.at[ids_v.at[pl.ds(G0, G1)]], rows_v.at[pl.ds(G0, G1)],
        sem_b).wait()


def _count_pad(ids_v, lanes):
    zacc = plsc.all_reduce_population_count(ids_v[pl.ds(0, LANES)] == 0)
    for k in range(1, 12):
        zacc = zacc + plsc.all_reduce_population_count(
            ids_v[pl.ds(k * LANES, LANES)] == 0)
    zacc = zacc + plsc.all_reduce_population_count(
        (ids_v[pl.ds(192, LANES)] == 0) & (lanes < 8))
    return zacc.astype(jnp.float32)


def _accumulate(rows_v, base, n, accs):
    def _add8(r0, accs):
        a0, a1, a2, a3 = accs
        for u in range(8):
            r = r0 + u
            a0 = a0 + rows_v[r, pl.ds(0, LANES)]
            a1 = a1 + rows_v[r, pl.ds(16, LANES)]
            a2 = a2 + rows_v[r, pl.ds(32, LANES)]
            a3 = a3 + rows_v[r, pl.ds(48, LANES)]
        return (a0, a1, a2, a3)

    def acc_body(j, accs):
        r0 = base + j * 16
        return _add8(r0 + 8, _add8(r0, accs))

    accs = lax.fori_loop(0, n // 16, acc_body, accs)
    if n % 16:
        accs = _add8(base + (n // 16) * 16, accs)
    return accs


def _tec_kernel(tok_hbm, tab_hbm, out_hbm, *scratch):
    ids6 = scratch[0:NIDS]
    rows4 = scratch[NIDS:NIDS + NG]
    e0_v = scratch[NIDS + NG]
    outs2 = scratch[NIDS + NG + 1:NIDS + NG + 3]
    sems = scratch[NIDS + NG + 3:]
    idsem6 = sems[0:NIDS]
    gsema = sems[NIDS:NIDS + NG]
    gsemb = sems[NIDS + NG:NIDS + 2 * NG]
    osem2 = sems[NIDS + 2 * NG:NIDS + 2 * NG + 2]

    wid = lax.axis_index("s") * NC + lax.axis_index("c")
    base = wid * ROWS_PER_W

    pltpu.sync_copy(tab_hbm.at[pl.ds(0, 1)], e0_v)
    lanes = lax.iota(jnp.int32, LANES)

    def row_step(i, m, do_ids, do_gather, do_outwait):
        b = base + i
        if do_ids:
            sl = (m + IDS_AHEAD) % NIDS
            _issue_ids(tok_hbm, b + IDS_AHEAD, ids6[sl], idsem6[sl])
        if do_gather:
            sli = (m + G_AHEAD) % NIDS
            slg = (m + G_AHEAD) % NG
            _wait_ids(tok_hbm, b + G_AHEAD, ids6[sli], idsem6[sli])
            _issue_gather(tab_hbm, ids6[sli], rows4[slg], gsema[slg],
                          gsemb[slg])
        n0v = _count_pad(ids6[m % NIDS], lanes)
        invv = 1.0 / jnp.maximum(float(L) - n0v, 1.0)
        z = jnp.zeros((LANES,), jnp.float32)
        _wait_gather_a(tab_hbm, ids6[m % NIDS], rows4[m % NG], gsema[m % NG])
        accs = _accumulate(rows4[m % NG], 0, G0, (z, z, z, z))
        _wait_gather_b(tab_hbm, ids6[m % NIDS], rows4[m % NG], gsemb[m % NG])
        accs = _accumulate(rows4[m % NG], G0, G1, accs)
        if do_outwait:
            pltpu.make_async_copy(
                outs2[m % 2], out_hbm.at[pl.ds((b - 2) * D, D)],
                osem2[m % 2]).wait()
        for c in range(4):
            e0c = e0_v[0, pl.ds(c * LANES, LANES)]
            outs2[m % 2][pl.ds(c * LANES, LANES)] = (accs[c] - n0v * e0c) * invv
        pltpu.async_copy(outs2[m % 2], out_hbm.at[pl.ds(b * D, D)], osem2[m % 2])

    for k in range(IDS_AHEAD):
        _issue_ids(tok_hbm, base + k, ids6[k], idsem6[k])
    for k in range(G_AHEAD):
        _wait_ids(tok_hbm, base + k, ids6[k], idsem6[k])
        _issue_gather(tab_hbm, ids6[k], rows4[k], gsema[k], gsemb[k])
    for k in range(G_AHEAD):
        row_step(k, k, True, True, k >= 2)

    def unroll_body(q, carry):
        for s in range(UNROLL):
            row_step(G_AHEAD + q * UNROLL + s, G_AHEAD + s, True, True, True)
        return carry

    lax.fori_loop(0, N_STEADY // UNROLL, unroll_body, 0)

    for i in range(TAIL_START, ROWS_PER_W):
        row_step(i, i % UNROLL, i + IDS_AHEAD < ROWS_PER_W,
                 i + G_AHEAD < ROWS_PER_W, True)
    last = base + ROWS_PER_W - 2
    pltpu.make_async_copy(
        outs2[0], out_hbm.at[pl.ds(last * D, D)], osem2[0]).wait()
    pltpu.make_async_copy(
        outs2[1], out_hbm.at[pl.ds((last + 1) * D, D)], osem2[1]).wait()


@functools.partial(
    pl.kernel,
    out_type=jax.ShapeDtypeStruct((B * D,), jnp.float32),
    mesh=plsc.VectorSubcoreMesh(core_axis_name="c", subcore_axis_name="s"),
    compiler_params=pltpu.CompilerParams(
        needs_layout_passes=False, use_tc_tiling_on_sc=False),
    scratch_types=(
        [pltpu.VMEM((208,), jnp.int32)] * NIDS
        + [pltpu.VMEM((L, D), jnp.float32)] * NG
        + [pltpu.VMEM((1, D), jnp.float32)]
        + [pltpu.VMEM((D,), jnp.float32)] * 2
        + [pltpu.SemaphoreType.DMA] * (NIDS + 2 * NG + 2)
    ),
)
def _sc_encode(*args):
    _tec_kernel(*args)


def kernel(token_ids, embed_weight):
    flat = _sc_encode(token_ids.astype(jnp.int32).reshape(-1), embed_weight)
    return flat.reshape(B, D)


# speedup vs baseline: 1.2220x; 1.2220x over previous
import functools

import jax
import jax.numpy as jnp
from jax import lax
from jax.experimental import pallas as pl
from jax.experimental.pallas import tpu as pltpu
from jax.experimental.pallas import tpu_sc as plsc

D = 64
B = 16384
L = 200
LANES = 16

NC = 2
NS = 16
NW = NC * NS
ROWS_PER_W = B // NW

G0, G1 = 96, 104

IDS_AHEAD, NIDS = 5, 6
G_AHEAD, NG = 3, 4
UNROLL = 12
N_STEADY = ((ROWS_PER_W - IDS_AHEAD - G_AHEAD) // UNROLL) * UNROLL
TAIL_START = G_AHEAD + N_STEADY


def _issue_ids(tok_hbm, b, ids_v, sem):
    pltpu.async_copy(tok_hbm.at[pl.ds(b * L, L)], ids_v.at[pl.ds(0, L)], sem)


def _wait_ids(tok_hbm, b, ids_v, sem):
    pltpu.make_async_copy(
        tok_hbm.at[pl.ds(b * L, L)], ids_v.at[pl.ds(0, L)], sem).wait()


def _issue_gather(tab_hbm, ids_v, rows_v, sem_a, sem_b):
    pltpu.async_copy(
        tab_hbm.at[ids_v.at[pl.ds(0, G0)]], rows_v.at[pl.ds(0, G0)], sem_a)
    pltpu.async_copy(
        tab_hbm.at[ids_v.at[pl.ds(G0, G1)]], rows_v.at[pl.ds(G0, G1)], sem_b)


def _wait_gather_a(tab_hbm, ids_v, rows_v, sem_a):
    pltpu.make_async_copy(
        tab_hbm.at[ids_v.at[pl.ds(0, G0)]], rows_v.at[pl.ds(0, G0)],
        sem_a).wait()


def _wait_gather_b(tab_hbm, ids_v, rows_v, sem_b):
    pltpu.make_async_copy(
        tab_hbm.at[ids_v.at[pl.ds(G0, G1)]], rows_v.at[pl.ds(G0, G1)],
        sem_b).wait()


def _count_pad(ids_v, lanes):
    zacc = plsc.all_reduce_population_count(ids_v[pl.ds(0, LANES)] == 0)
    for k in range(1, 12):
        zacc = zacc + plsc.all_reduce_population_count(
            ids_v[pl.ds(k * LANES, LANES)] == 0)
    zacc = zacc + plsc.all_reduce_population_count(
        (ids_v[pl.ds(192, LANES)] == 0) & (lanes < 8))
    return zacc.astype(jnp.float32)


def _accumulate(rows_v, base, n, accs):
    def acc_body(j, accs):
        a0, a1, a2, a3 = accs
        r0 = base + j * 8
        for u in range(8):
            r = r0 + u
            a0 = a0 + rows_v[r, pl.ds(0, LANES)]
            a1 = a1 + rows_v[r, pl.ds(16, LANES)]
            a2 = a2 + rows_v[r, pl.ds(32, LANES)]
            a3 = a3 + rows_v[r, pl.ds(48, LANES)]
        return (a0, a1, a2, a3)

    return lax.fori_loop(0, n // 8, acc_body, accs)


def _tec_kernel(tok_hbm, tab_hbm, out_hbm, *scratch):
    ids6 = scratch[0:NIDS]
    rows4 = scratch[NIDS:NIDS + NG]
    e0_v = scratch[NIDS + NG]
    outs2 = scratch[NIDS + NG + 1:NIDS + NG + 3]
    sems = scratch[NIDS + NG + 3:]
    idsem6 = sems[0:NIDS]
    gsema = sems[NIDS:NIDS + NG]
    gsemb = sems[NIDS + NG:NIDS + 2 * NG]
    osem2 = sems[NIDS + 2 * NG:NIDS + 2 * NG + 2]

    wid = lax.axis_index("s") * NC + lax.axis_index("c")
    base = wid * ROWS_PER_W

    pltpu.sync_copy(tab_hbm.at[pl.ds(0, 1)], e0_v)
    lanes = lax.iota(jnp.int32, LANES)

    def row_step(i, m, do_ids, do_gather, do_outwait):
        b = base + i
        if do_ids:
            sl = (m + IDS_AHEAD) % NIDS
            _issue_ids(tok_hbm, b + IDS_AHEAD, ids6[sl], idsem6[sl])
        if do_gather:
            sli = (m + G_AHEAD) % NIDS
            slg = (m + G_AHEAD) % NG
            _wait_ids(tok_hbm, b + G_AHEAD, ids6[sli], idsem6[sli])
            _issue_gather(tab_hbm, ids6[sli], rows4[slg], gsema[slg],
                          gsemb[slg])
        n0v = _count_pad(ids6[m % NIDS], lanes)
        invv = 1.0 / jnp.maximum(float(L) - n0v, 1.0)
        z = jnp.zeros((LANES,), jnp.float32)
        _wait_gather_a(tab_hbm, ids6[m % NIDS], rows4[m % NG], gsema[m % NG])
        accs = _accumulate(rows4[m % NG], 0, G0, (z, z, z, z))
        _wait_gather_b(tab_hbm, ids6[m % NIDS], rows4[m % NG], gsemb[m % NG])
        accs = _accumulate(rows4[m % NG], G0, G1, accs)
        if do_outwait:
            pltpu.make_async_copy(
                outs2[m % 2], out_hbm.at[pl.ds((b - 2) * D, D)],
                osem2[m % 2]).wait()
        for c in range(4):
            e0c = e0_v[0, pl.ds(c * LANES, LANES)]
            outs2[m % 2][pl.ds(c * LANES, LANES)] = (accs[c] - n0v * e0c) * invv
        pltpu.async_copy(outs2[m % 2], out_hbm.at[pl.ds(b * D, D)], osem2[m % 2])

    for k in range(IDS_AHEAD):
        _issue_ids(tok_hbm, base + k, ids6[k], idsem6[k])
    for k in range(G_AHEAD):
        _wait_ids(tok_hbm, base + k, ids6[k], idsem6[k])
        _issue_gather(tab_hbm, ids6[k], rows4[k], gsema[k], gsemb[k])
    for k in range(G_AHEAD):
        row_step(k, k, True, True, k >= 2)

    def unroll_body(q, carry):
        for s in range(UNROLL):
            row_step(G_AHEAD + q * UNROLL + s, G_AHEAD + s, True, True, True)
        return carry

    lax.fori_loop(0, N_STEADY // UNROLL, unroll_body, 0)

    for i in range(TAIL_START, ROWS_PER_W):
        row_step(i, i % UNROLL, i + IDS_AHEAD < ROWS_PER_W,
                 i + G_AHEAD < ROWS_PER_W, True)
    last = base + ROWS_PER_W - 2
    pltpu.make_async_copy(
        outs2[0], out_hbm.at[pl.ds(last * D, D)], osem2[0]).wait()
    pltpu.make_async_copy(
        outs2[1], out_hbm.at[pl.ds((last + 1) * D, D)], osem2[1]).wait()


@functools.partial(
    pl.kernel,
    out_type=jax.ShapeDtypeStruct((B * D,), jnp.float32),
    mesh=plsc.VectorSubcoreMesh(core_axis_name="c", subcore_axis_name="s"),
    compiler_params=pltpu.CompilerParams(
        needs_layout_passes=False, use_tc_tiling_on_sc=False),
    scratch_types=(
        [pltpu.VMEM((208,), jnp.int32)] * NIDS
        + [pltpu.VMEM((L, D), jnp.float32)] * NG
        + [pltpu.VMEM((1, D), jnp.float32)]
        + [pltpu.VMEM((D,), jnp.float32)] * 2
        + [pltpu.SemaphoreType.DMA] * (NIDS + 2 * NG + 2)
    ),
)
def _sc_encode(*args):
    _tec_kernel(*args)


def kernel(token_ids, embed_weight):
    flat = _sc_encode(token_ids.astype(jnp.int32).reshape(-1), embed_weight)
    return flat.reshape(B, D)
